# baseline (speedup 1.0000x reference)
import math
from functools import partial

import jax
import jax.numpy as jnp
from jax.experimental import pallas as pl
from jax.experimental.pallas import tpu as pltpu

_EPS = 1e-07
_LANE = 128
_TR = 1024


def _partial_kernel(x_ref, t_ref, inter_ref, card_ref, *, tr):
    k = pl.program_id(1)

    @pl.when(k == 0)
    def _():
        inter_ref[...] = jnp.zeros_like(inter_ref)
        card_ref[...] = jnp.zeros_like(card_ref)

    x = x_ref[...]
    t = t_ref[...]
    bsz = x.shape[0]
    prod = (x * t).reshape(bsz, tr // 8, 8, _LANE)
    card = (x + t).reshape(bsz, tr // 8, 8, _LANE)
    inter_ref[...] += jnp.sum(prod, axis=1)[None]
    card_ref[...] += jnp.sum(card, axis=1)[None]


def kernel(x, target):
    b = x.shape[0]
    n = math.prod(x.shape[1:])
    r = n // _LANE
    tr = _TR
    kb = r // tr
    kpp = kb // 2

    x3 = x.reshape(b, r, _LANE)
    t3 = target.reshape(b, r, _LANE)

    in_spec = pl.BlockSpec((b, tr, _LANE), lambda p, k: (0, p * kpp + k, 0))
    out_spec = pl.BlockSpec((1, b, 8, _LANE), lambda p, k: (p, 0, 0, 0))

    inter_p, card_p = pl.pallas_call(
        partial(_partial_kernel, tr=tr),
        out_shape=(jax.ShapeDtypeStruct((2, b, 8, _LANE), jnp.float32),
                   jax.ShapeDtypeStruct((2, b, 8, _LANE), jnp.float32)),
        grid=(2, kpp),
        in_specs=[in_spec, in_spec],
        out_specs=(out_spec, out_spec),
        compiler_params=pltpu.CompilerParams(
            dimension_semantics=("parallel", "arbitrary"),
            vmem_limit_bytes=52 * 1024 * 1024,
        ),
    )(x3, t3)

    inter = jnp.sum(inter_p.reshape(2 * b, -1), axis=1).reshape(2, b).sum(0)
    card = jnp.sum(card_p.reshape(2 * b, -1), axis=1).reshape(2, b).sum(0)
    dice = 1.0 - 2.0 * inter / (card + _EPS)
    max_val = jnp.max(dice)
    weights = dice / max_val
    return jnp.mean(max_val * weights)


# speedup vs baseline: 2.7608x; 2.7608x over previous
from functools import partial

import jax
import jax.numpy as jnp
from jax.experimental import pallas as pl
from jax.experimental.pallas import tpu as pltpu

_EPS = 1e-07
_DD = 4


def _partial_kernel(x_ref, t_ref, inter_ref, card_ref, *, c, dd):
    k = pl.program_id(0)

    @pl.when(k == 0)
    def _():
        inter_ref[...] = jnp.zeros_like(inter_ref)
        card_ref[...] = jnp.zeros_like(card_ref)

    x = x_ref[...]
    t = t_ref[...]
    b = x.shape[0]
    prod = (x * t).reshape(b, c * dd, 64, 64)
    card = (x + t).reshape(b, c * dd, 64, 64)
    inter_ref[...] += jnp.sum(prod, axis=1)
    card_ref[...] += jnp.sum(card, axis=1)


def kernel(x, target):
    b, c, d, h, w = x.shape
    dd = _DD
    kb = d // dd

    in_spec = pl.BlockSpec((b, c, dd, h, w), lambda k: (0, 0, k, 0, 0))
    out_spec = pl.BlockSpec((b, h, w), lambda k: (0, 0, 0))

    inter_p, card_p = pl.pallas_call(
        partial(_partial_kernel, c=c, dd=dd),
        out_shape=(jax.ShapeDtypeStruct((b, h, w), jnp.float32),
                   jax.ShapeDtypeStruct((b, h, w), jnp.float32)),
        grid=(kb,),
        in_specs=[in_spec, in_spec],
        out_specs=(out_spec, out_spec),
        compiler_params=pltpu.CompilerParams(
            vmem_limit_bytes=52 * 1024 * 1024,
        ),
    )(x, target)

    inter = jnp.sum(inter_p.reshape(b, -1), axis=1)
    card = jnp.sum(card_p.reshape(b, -1), axis=1)
    dice = 1.0 - 2.0 * inter / (card + _EPS)
    max_val = jnp.max(dice)
    weights = dice / max_val
    return jnp.mean(max_val * weights)


# speedup vs baseline: 2.7694x; 1.0031x over previous
from functools import partial

import jax
import jax.numpy as jnp
from jax.experimental import pallas as pl
from jax.experimental.pallas import tpu as pltpu

_EPS = 1e-07
_DD = 4


def _partial_kernel(x_ref, t_ref, inter_ref, card_ref, *, c, dd):
    k = pl.program_id(1)

    @pl.when(k == 0)
    def _():
        inter_ref[...] = jnp.zeros_like(inter_ref)
        card_ref[...] = jnp.zeros_like(card_ref)

    x = x_ref[...]
    t = t_ref[...]
    b = x.shape[0]
    prod = (x * t).reshape(b, c * dd, 64, 64)
    card = (x + t).reshape(b, c * dd, 64, 64)
    inter_ref[...] += jnp.sum(prod, axis=1)[None]
    card_ref[...] += jnp.sum(card, axis=1)[None]


def kernel(x, target):
    b, c, d, h, w = x.shape
    dd = _DD
    kb = d // dd
    kpp = kb // 2

    in_spec = pl.BlockSpec((b, c, dd, h, w),
                           lambda p, k: (0, 0, p * kpp + k, 0, 0))
    out_spec = pl.BlockSpec((1, b, h, w), lambda p, k: (p, 0, 0, 0))

    inter_p, card_p = pl.pallas_call(
        partial(_partial_kernel, c=c, dd=dd),
        out_shape=(jax.ShapeDtypeStruct((2, b, h, w), jnp.float32),
                   jax.ShapeDtypeStruct((2, b, h, w), jnp.float32)),
        grid=(2, kpp),
        in_specs=[in_spec, in_spec],
        out_specs=(out_spec, out_spec),
        compiler_params=pltpu.CompilerParams(
            dimension_semantics=("parallel", "arbitrary"),
            vmem_limit_bytes=52 * 1024 * 1024,
        ),
    )(x, target)

    inter = jnp.sum(inter_p.reshape(2, b, -1), axis=(0, 2))
    card = jnp.sum(card_p.reshape(2, b, -1), axis=(0, 2))
    dice = 1.0 - 2.0 * inter / (card + _EPS)
    max_val = jnp.max(dice)
    weights = dice / max_val
    return jnp.mean(max_val * weights)
